# baseline (speedup 1.0000x reference)
import jax
import jax.numpy as jnp
from jax.experimental import pallas as pl
from jax.experimental.pallas import tpu as pltpu

B, D, C = 2048, 1024, 8192
BC = 512
S = 2
NC = C // (BC * S)

_LN2 = 0.6931471805599453
_INV_LN2 = 1.4426950408889634


def _main_kernel(x_ref, cen_a_ref, cen_b_ref, y_ref, loss_ref, corr_ref,
                 xs_ref, m_ref, l_ref, t_ref, a_ref):
    c = pl.program_id(0)

    @pl.when(c == 0)
    def _init():
        xs_ref[...] = x_ref[...].T * (2.0 * _INV_LN2)
        m_ref[...] = jnp.full(m_ref.shape, -jnp.inf, dtype=jnp.float32)
        l_ref[...] = jnp.zeros(l_ref.shape, dtype=jnp.float32)
        t_ref[...] = jnp.zeros(t_ref.shape, dtype=jnp.float32)
        a_ref[...] = jnp.zeros(a_ref.shape, dtype=jnp.float32)

    def _chunk(cb, base):
        acc = jnp.dot(cb, xs_ref[...], preferred_element_type=jnp.float32)
        c2h = jnp.sum(cb * cb, axis=1, keepdims=True) * _INV_LN2
        h = acc - c2h

        row = jax.lax.broadcasted_iota(jnp.int32, (BC, B), 0)

        v, idx = h, row
        k = BC
        while k > 8:
            k //= 2
            gt = v[:k] >= v[k:]
            v = jnp.where(gt, v[:k], v[k:])
            idx = jnp.where(gt, idx[:k], idx[k:])
        cmax = jnp.max(v, axis=0, keepdims=True)
        camax_loc = jnp.min(jnp.where(v >= cmax, idx, C), axis=0,
                            keepdims=True)

        m_old = jnp.max(m_ref[...], axis=0, keepdims=True)
        l_old = jnp.max(l_ref[...], axis=0, keepdims=True)
        a_old = jnp.max(a_ref[...], axis=0, keepdims=True)

        m_new = jnp.maximum(m_old, cmax)
        p = jnp.exp2(h - m_new)
        p_sum = jnp.sum(p, axis=0, keepdims=True)
        l_new = l_old * jnp.exp2(m_old - m_new) + p_sum

        camax = (camax_loc + base).astype(jnp.float32)
        a_new = jnp.where(cmax > m_old, camax, a_old)

        yloc = y_ref[0] - base
        tsum = jnp.sum(jnp.where(row == yloc, h, 0.0),
                       axis=0, keepdims=True)

        m_ref[...] = jnp.broadcast_to(m_new, m_ref.shape)
        l_ref[...] = jnp.broadcast_to(l_new, l_ref.shape)
        a_ref[...] = jnp.broadcast_to(a_new, a_ref.shape)
        t_ref[...] = t_ref[...] + jnp.broadcast_to(tsum, t_ref.shape)

    base0 = c * (S * BC)
    _chunk(cen_a_ref[...], base0)
    _chunk(cen_b_ref[...], base0 + BC)

    @pl.when(c == NC - 1)
    def _fin():
        m_c = jnp.max(m_ref[...], axis=0, keepdims=True)
        l_c = jnp.max(l_ref[...], axis=0, keepdims=True)
        t_c = jnp.max(t_ref[...], axis=0, keepdims=True)
        a_c = jnp.max(a_ref[...], axis=0, keepdims=True)
        loss_row = (m_c + jnp.log2(l_c) - t_c) * _LN2
        corr_row = (a_c == y_ref[0].astype(jnp.float32)).astype(jnp.float32)
        ls = jnp.sum(loss_row, keepdims=True)
        cs = jnp.sum(corr_row, keepdims=True)
        loss_ref[...] = jnp.broadcast_to(ls, (8, 128))
        corr_ref[...] = jnp.broadcast_to(cs, (8, 128))


@jax.jit
def kernel(x, y, centroids):
    y3 = y.astype(jnp.int32).reshape(1, 1, B)
    loss_t, corr_t = pl.pallas_call(
        _main_kernel,
        grid=(NC,),
        in_specs=[
            pl.BlockSpec((B, D), lambda c: (0, 0)),
            pl.BlockSpec((BC, D), lambda c: (c * S, 0)),
            pl.BlockSpec((BC, D), lambda c: (c * S + 1, 0)),
            pl.BlockSpec((1, 1, B), lambda c: (0, 0, 0)),
        ],
        out_specs=(pl.BlockSpec((8, 128), lambda c: (0, 0)),
                   pl.BlockSpec((8, 128), lambda c: (0, 0))),
        out_shape=(jax.ShapeDtypeStruct((8, 128), jnp.float32),
                   jax.ShapeDtypeStruct((8, 128), jnp.float32)),
        scratch_shapes=[
            pltpu.VMEM((D, B), jnp.float32),
            pltpu.VMEM((8, B), jnp.float32),
            pltpu.VMEM((8, B), jnp.float32),
            pltpu.VMEM((8, B), jnp.float32),
            pltpu.VMEM((8, B), jnp.float32),
        ],
        compiler_params=pltpu.CompilerParams(
            dimension_semantics=("arbitrary",),
            vmem_limit_bytes=100 * 1024 * 1024,
        ),
    )(x, centroids, centroids, y3)
    loss = loss_t[0, 0] / B
    score = corr_t[0, 0] / B
    return loss, score


# speedup vs baseline: 1.0027x; 1.0027x over previous
import jax
import jax.numpy as jnp
from jax.experimental import pallas as pl
from jax.experimental.pallas import tpu as pltpu

B, D, C = 2048, 1024, 8192
BC = 512
S = 2
NC = C // (BC * S)

_LN2 = 0.6931471805599453
_INV_LN2 = 1.4426950408889634


def _main_kernel(x0_ref, x1_ref, x2_ref, x3_ref, cen_a_ref, cen_b_ref, y_ref,
                 loss_ref, corr_ref, xs_ref, m_ref, l_ref, t_ref, a_ref):
    c = pl.program_id(0)
    DQ = D // 4

    @pl.when(c == 0)
    def _init():
        for j, xr in enumerate((x0_ref, x1_ref, x2_ref, x3_ref)):
            xs_ref[j * DQ:(j + 1) * DQ, :] = xr[...].T * (2.0 * _INV_LN2)
        m_ref[...] = jnp.full(m_ref.shape, -jnp.inf, dtype=jnp.float32)
        l_ref[...] = jnp.zeros(l_ref.shape, dtype=jnp.float32)
        t_ref[...] = jnp.zeros(t_ref.shape, dtype=jnp.float32)
        a_ref[...] = jnp.zeros(a_ref.shape, dtype=jnp.float32)

    def _chunk(cb, base):
        acc = jnp.dot(cb, xs_ref[...], preferred_element_type=jnp.float32)
        c2h = jnp.sum(cb * cb, axis=1, keepdims=True) * _INV_LN2
        h = acc - c2h

        cmax = jnp.max(h, axis=0, keepdims=True)

        m_old = jnp.max(m_ref[...], axis=0, keepdims=True)
        l_old = jnp.max(l_ref[...], axis=0, keepdims=True)
        a_old = jnp.max(a_ref[...], axis=0, keepdims=True)

        m_new = jnp.maximum(m_old, cmax)
        p = jnp.exp2(h - m_new)
        p_sum = jnp.sum(p, axis=0, keepdims=True)
        l_new = l_old * jnp.exp2(m_old - m_new) + p_sum

        row = jax.lax.broadcasted_iota(jnp.int32, (BC, B), 0)
        camax = (jnp.min(jnp.where(p >= 1.0, row, C), axis=0,
                         keepdims=True) + base).astype(jnp.float32)
        a_new = jnp.where(cmax > m_old, camax, a_old)

        yloc = y_ref[0] - base
        tsum = jnp.sum(jnp.where(row == yloc, h, 0.0),
                       axis=0, keepdims=True)

        m_ref[...] = jnp.broadcast_to(m_new, m_ref.shape)
        l_ref[...] = jnp.broadcast_to(l_new, l_ref.shape)
        a_ref[...] = jnp.broadcast_to(a_new, a_ref.shape)
        t_ref[...] = t_ref[...] + jnp.broadcast_to(tsum, t_ref.shape)

    base0 = c * (S * BC)
    _chunk(cen_a_ref[...], base0)
    _chunk(cen_b_ref[...], base0 + BC)

    @pl.when(c == NC - 1)
    def _fin():
        m_c = jnp.max(m_ref[...], axis=0, keepdims=True)
        l_c = jnp.max(l_ref[...], axis=0, keepdims=True)
        t_c = jnp.max(t_ref[...], axis=0, keepdims=True)
        a_c = jnp.max(a_ref[...], axis=0, keepdims=True)
        loss_row = (m_c + jnp.log2(l_c) - t_c) * _LN2
        corr_row = (a_c == y_ref[0].astype(jnp.float32)).astype(jnp.float32)
        ls = jnp.sum(loss_row, keepdims=True)
        cs = jnp.sum(corr_row, keepdims=True)
        loss_ref[...] = jnp.broadcast_to(ls, (8, 128))
        corr_ref[...] = jnp.broadcast_to(cs, (8, 128))


@jax.jit
def kernel(x, y, centroids):
    y3 = y.astype(jnp.int32).reshape(1, 1, B)
    loss_t, corr_t = pl.pallas_call(
        _main_kernel,
        grid=(NC,),
        in_specs=[
            pl.BlockSpec((B, D // 4), lambda c: (0, 0)),
            pl.BlockSpec((B, D // 4), lambda c: (0, 1)),
            pl.BlockSpec((B, D // 4), lambda c: (0, 2)),
            pl.BlockSpec((B, D // 4), lambda c: (0, 3)),
            pl.BlockSpec((BC, D), lambda c: (c * S, 0)),
            pl.BlockSpec((BC, D), lambda c: (c * S + 1, 0)),
            pl.BlockSpec((1, 1, B), lambda c: (0, 0, 0)),
        ],
        out_specs=(pl.BlockSpec((8, 128), lambda c: (0, 0)),
                   pl.BlockSpec((8, 128), lambda c: (0, 0))),
        out_shape=(jax.ShapeDtypeStruct((8, 128), jnp.float32),
                   jax.ShapeDtypeStruct((8, 128), jnp.float32)),
        scratch_shapes=[
            pltpu.VMEM((D, B), jnp.float32),
            pltpu.VMEM((8, B), jnp.float32),
            pltpu.VMEM((8, B), jnp.float32),
            pltpu.VMEM((8, B), jnp.float32),
            pltpu.VMEM((8, B), jnp.float32),
        ],
        compiler_params=pltpu.CompilerParams(
            dimension_semantics=("arbitrary",),
            vmem_limit_bytes=100 * 1024 * 1024,
        ),
    )(x, x, x, x, centroids, centroids, y3)
    loss = loss_t[0, 0] / B
    score = corr_t[0, 0] / B
    return loss, score


# speedup vs baseline: 1.1088x; 1.1059x over previous
import jax
import jax.numpy as jnp
from jax.experimental import pallas as pl
from jax.experimental.pallas import tpu as pltpu

B, D, C = 2048, 1024, 8192
BC = 512
S = 2
NC = C // (BC * S)

_LN2 = 0.6931471805599453
_INV_LN2 = 1.4426950408889634


def _main_kernel(x_ref, cen_a_ref, cen_b_ref, y_ref, loss_ref, corr_ref,
                 xs_ref, m_ref, l_ref, t_ref):
    c = pl.program_id(0)

    @pl.when(c == 0)
    def _init():
        xs_ref[...] = x_ref[...].T * (2.0 * _INV_LN2)
        m_ref[...] = jnp.full(m_ref.shape, -jnp.inf, dtype=jnp.float32)
        l_ref[...] = jnp.zeros(l_ref.shape, dtype=jnp.float32)
        t_ref[...] = jnp.zeros(t_ref.shape, dtype=jnp.float32)

    def _chunk(cb, base):
        acc = jnp.dot(cb, xs_ref[...], preferred_element_type=jnp.float32)
        c2h = jnp.sum(cb * cb, axis=1, keepdims=True) * _INV_LN2
        h = acc - c2h

        cmax = jnp.max(h, axis=0, keepdims=True)

        m_old = jnp.max(m_ref[...], axis=0, keepdims=True)
        l_old = jnp.max(l_ref[...], axis=0, keepdims=True)

        m_new = jnp.maximum(m_old, cmax)
        p = jnp.exp2(h - m_new)
        p_sum = jnp.sum(p, axis=0, keepdims=True)
        l_new = l_old * jnp.exp2(m_old - m_new) + p_sum

        row = jax.lax.broadcasted_iota(jnp.int32, (BC, B), 0)
        yloc = y_ref[0] - base
        tsum = jnp.sum(jnp.where(row == yloc, h, 0.0),
                       axis=0, keepdims=True)

        m_ref[...] = jnp.broadcast_to(m_new, m_ref.shape)
        l_ref[...] = jnp.broadcast_to(l_new, l_ref.shape)
        t_ref[...] = t_ref[...] + jnp.broadcast_to(tsum, t_ref.shape)

    base0 = c * (S * BC)
    _chunk(cen_a_ref[...], base0)
    _chunk(cen_b_ref[...], base0 + BC)

    @pl.when(c == NC - 1)
    def _fin():
        m_c = jnp.max(m_ref[...], axis=0, keepdims=True)
        l_c = jnp.max(l_ref[...], axis=0, keepdims=True)
        t_c = jnp.max(t_ref[...], axis=0, keepdims=True)
        loss_row = (m_c + jnp.log2(l_c) - t_c) * _LN2
        corr_row = (t_c == m_c).astype(jnp.float32)
        ls = jnp.sum(loss_row, keepdims=True)
        cs = jnp.sum(corr_row, keepdims=True)
        loss_ref[...] = jnp.broadcast_to(ls, (8, 128))
        corr_ref[...] = jnp.broadcast_to(cs, (8, 128))


@jax.jit
def kernel(x, y, centroids):
    y3 = y.astype(jnp.int32).reshape(1, 1, B)
    loss_t, corr_t = pl.pallas_call(
        _main_kernel,
        grid=(NC,),
        in_specs=[
            pl.BlockSpec((B, D), lambda c: (0, 0)),
            pl.BlockSpec((BC, D), lambda c: (c * S, 0)),
            pl.BlockSpec((BC, D), lambda c: (c * S + 1, 0)),
            pl.BlockSpec((1, 1, B), lambda c: (0, 0, 0)),
        ],
        out_specs=(pl.BlockSpec((8, 128), lambda c: (0, 0)),
                   pl.BlockSpec((8, 128), lambda c: (0, 0))),
        out_shape=(jax.ShapeDtypeStruct((8, 128), jnp.float32),
                   jax.ShapeDtypeStruct((8, 128), jnp.float32)),
        scratch_shapes=[
            pltpu.VMEM((D, B), jnp.float32),
            pltpu.VMEM((8, B), jnp.float32),
            pltpu.VMEM((8, B), jnp.float32),
            pltpu.VMEM((8, B), jnp.float32),
        ],
        compiler_params=pltpu.CompilerParams(
            dimension_semantics=("arbitrary",),
            vmem_limit_bytes=100 * 1024 * 1024,
        ),
    )(x, centroids, centroids, y3)
    loss = loss_t[0, 0] / B
    score = corr_t[0, 0] / B
    return loss, score


# speedup vs baseline: 1.2069x; 1.0884x over previous
import jax
import jax.numpy as jnp
from jax.experimental import pallas as pl
from jax.experimental.pallas import tpu as pltpu

B, D, C = 2048, 1024, 8192
BC = 512
S = 2
NC = C // (BC * S)

_LN2 = 0.6931471805599453
_INV_LN2 = 1.4426950408889634


def _main_kernel(x_ref, cen_a_ref, cen_b_ref, y_ref, loss_ref, corr_ref,
                 xs_ref, m_ref, l_ref, t_ref):
    c = pl.program_id(0)

    @pl.when(c == 0)
    def _init():
        xs_ref[...] = x_ref[...].T * (2.0 * _INV_LN2)
        m_ref[...] = jnp.full(m_ref.shape, -jnp.inf, dtype=jnp.float32)
        l_ref[...] = jnp.zeros(l_ref.shape, dtype=jnp.float32)
        t_ref[...] = jnp.zeros(t_ref.shape, dtype=jnp.float32)

    def _chunk(cb, base):
        acc = jnp.dot(cb, xs_ref[...], preferred_element_type=jnp.float32)
        c2h = jnp.sum(cb * cb, axis=1, keepdims=True) * _INV_LN2
        h = acc - c2h

        cmax = jnp.max(h, axis=0, keepdims=True)

        m_old = jnp.max(m_ref[...], axis=0, keepdims=True)
        l_old = jnp.max(l_ref[...], axis=0, keepdims=True)

        m_new = jnp.maximum(m_old, cmax)
        p = jnp.exp2(h - m_new)
        p_sum = jnp.sum(p, axis=0, keepdims=True)
        l_new = l_old * jnp.exp2(m_old - m_new) + p_sum

        row = jax.lax.broadcasted_iota(jnp.int32, (BC, B), 0)
        yloc = y_ref[0] - base
        tsum = jnp.sum(jnp.where(row == yloc, h, 0.0),
                       axis=0, keepdims=True)

        m_ref[...] = jnp.broadcast_to(m_new, m_ref.shape)
        l_ref[...] = jnp.broadcast_to(l_new, l_ref.shape)
        t_ref[...] = t_ref[...] + jnp.broadcast_to(tsum, t_ref.shape)

    base0 = c * (S * BC)
    _chunk(cen_a_ref[...], base0)
    _chunk(cen_b_ref[...], base0 + BC)

    @pl.when(c == NC - 1)
    def _fin():
        m_c = jnp.max(m_ref[...], axis=0, keepdims=True)
        l_c = jnp.max(l_ref[...], axis=0, keepdims=True)
        t_c = jnp.max(t_ref[...], axis=0, keepdims=True)
        loss_row = (m_c + jnp.log2(l_c) - t_c) * _LN2
        corr_row = (t_c == m_c).astype(jnp.float32)
        loss_ref[0] = jnp.sum(loss_row) * (1.0 / B)
        corr_ref[0] = jnp.sum(corr_row) * (1.0 / B)


@jax.jit
def kernel(x, y, centroids):
    y3 = y.astype(jnp.int32).reshape(1, 1, B)
    loss_t, corr_t = pl.pallas_call(
        _main_kernel,
        grid=(NC,),
        in_specs=[
            pl.BlockSpec((B, D), lambda c: (0, 0)),
            pl.BlockSpec((BC, D), lambda c: (c * S, 0)),
            pl.BlockSpec((BC, D), lambda c: (c * S + 1, 0)),
            pl.BlockSpec((1, 1, B), lambda c: (0, 0, 0)),
        ],
        out_specs=(pl.BlockSpec(memory_space=pltpu.SMEM),
                   pl.BlockSpec(memory_space=pltpu.SMEM)),
        out_shape=(jax.ShapeDtypeStruct((1,), jnp.float32),
                   jax.ShapeDtypeStruct((1,), jnp.float32)),
        scratch_shapes=[
            pltpu.VMEM((D, B), jnp.float32),
            pltpu.VMEM((8, B), jnp.float32),
            pltpu.VMEM((8, B), jnp.float32),
            pltpu.VMEM((8, B), jnp.float32),
        ],
        compiler_params=pltpu.CompilerParams(
            dimension_semantics=("arbitrary",),
            vmem_limit_bytes=100 * 1024 * 1024,
        ),
    )(x, centroids, centroids, y3)
    return loss_t[0], corr_t[0]


# speedup vs baseline: 1.2303x; 1.0194x over previous
import jax
import jax.numpy as jnp
from jax.experimental import pallas as pl
from jax.experimental.pallas import tpu as pltpu

B, D, C = 2048, 1024, 8192
BC = 1024
S = 2
NC = C // (BC * S)

_LN2 = 0.6931471805599453
_INV_LN2 = 1.4426950408889634


def _main_kernel(x_ref, cen_a_ref, cen_b_ref, y_ref, loss_ref, corr_ref,
                 xs_ref, m_ref, l_ref, t_ref):
    c = pl.program_id(0)

    @pl.when(c == 0)
    def _init():
        xs_ref[...] = x_ref[...].T * (2.0 * _INV_LN2)
        m_ref[...] = jnp.full(m_ref.shape, -jnp.inf, dtype=jnp.float32)
        l_ref[...] = jnp.zeros(l_ref.shape, dtype=jnp.float32)
        t_ref[...] = jnp.zeros(t_ref.shape, dtype=jnp.float32)

    def _chunk(cb, base):
        acc = jnp.dot(cb, xs_ref[...], preferred_element_type=jnp.float32)
        c2h = jnp.sum(cb * cb, axis=1, keepdims=True) * _INV_LN2
        h = acc - c2h

        cmax = jnp.max(h, axis=0, keepdims=True)

        m_old = jnp.max(m_ref[...], axis=0, keepdims=True)
        l_old = jnp.max(l_ref[...], axis=0, keepdims=True)

        m_new = jnp.maximum(m_old, cmax)
        p = jnp.exp2(h - m_new)
        p_sum = jnp.sum(p, axis=0, keepdims=True)
        l_new = l_old * jnp.exp2(m_old - m_new) + p_sum

        row = jax.lax.broadcasted_iota(jnp.int32, (BC, B), 0)
        yloc = y_ref[0] - base
        tsum = jnp.sum(jnp.where(row == yloc, h, 0.0),
                       axis=0, keepdims=True)

        m_ref[...] = jnp.broadcast_to(m_new, m_ref.shape)
        l_ref[...] = jnp.broadcast_to(l_new, l_ref.shape)
        t_ref[...] = t_ref[...] + jnp.broadcast_to(tsum, t_ref.shape)

    base0 = c * (S * BC)
    _chunk(cen_a_ref[...], base0)
    _chunk(cen_b_ref[...], base0 + BC)

    @pl.when(c == NC - 1)
    def _fin():
        m_c = jnp.max(m_ref[...], axis=0, keepdims=True)
        l_c = jnp.max(l_ref[...], axis=0, keepdims=True)
        t_c = jnp.max(t_ref[...], axis=0, keepdims=True)
        loss_row = (m_c + jnp.log2(l_c) - t_c) * _LN2
        corr_row = (t_c == m_c).astype(jnp.float32)
        loss_ref[0] = jnp.sum(loss_row) * (1.0 / B)
        corr_ref[0] = jnp.sum(corr_row) * (1.0 / B)


@jax.jit
def kernel(x, y, centroids):
    y3 = y.astype(jnp.int32).reshape(1, 1, B)
    loss_t, corr_t = pl.pallas_call(
        _main_kernel,
        grid=(NC,),
        in_specs=[
            pl.BlockSpec((B, D), lambda c: (0, 0)),
            pl.BlockSpec((BC, D), lambda c: (c * S, 0)),
            pl.BlockSpec((BC, D), lambda c: (c * S + 1, 0)),
            pl.BlockSpec((1, 1, B), lambda c: (0, 0, 0)),
        ],
        out_specs=(pl.BlockSpec(memory_space=pltpu.SMEM),
                   pl.BlockSpec(memory_space=pltpu.SMEM)),
        out_shape=(jax.ShapeDtypeStruct((1,), jnp.float32),
                   jax.ShapeDtypeStruct((1,), jnp.float32)),
        scratch_shapes=[
            pltpu.VMEM((D, B), jnp.float32),
            pltpu.VMEM((8, B), jnp.float32),
            pltpu.VMEM((8, B), jnp.float32),
            pltpu.VMEM((8, B), jnp.float32),
        ],
        compiler_params=pltpu.CompilerParams(
            dimension_semantics=("arbitrary",),
            vmem_limit_bytes=100 * 1024 * 1024,
        ),
    )(x, centroids, centroids, y3)
    return loss_t[0], corr_t[0]


# speedup vs baseline: 1.2335x; 1.0027x over previous
import jax
import jax.numpy as jnp
from jax.experimental import pallas as pl
from jax.experimental.pallas import tpu as pltpu

B, D, C = 2048, 1024, 8192
BC = 1024
S = 2
NC = C // (BC * S)

_LN2 = 0.6931471805599453
_INV_LN2 = 1.4426950408889634


def _main_kernel(x0_ref, x1_ref, x2_ref, x3_ref, cen_a_ref, cen_b_ref, y_ref,
                 loss_ref, corr_ref, xs_ref, m_ref, l_ref, t_ref):
    c = pl.program_id(0)
    DQ = D // 4

    @pl.when(c == 0)
    def _init():
        for j, xr in enumerate((x0_ref, x1_ref, x2_ref, x3_ref)):
            xs_ref[j * DQ:(j + 1) * DQ, :] = xr[...].T * (2.0 * _INV_LN2)
        m_ref[...] = jnp.full(m_ref.shape, -jnp.inf, dtype=jnp.float32)
        l_ref[...] = jnp.zeros(l_ref.shape, dtype=jnp.float32)
        t_ref[...] = jnp.zeros(t_ref.shape, dtype=jnp.float32)

    def _chunk(cb, base):
        acc = jnp.dot(cb, xs_ref[...], preferred_element_type=jnp.float32)
        c2h = jnp.sum(cb * cb, axis=1, keepdims=True) * _INV_LN2
        h = acc - c2h

        cmax = jnp.max(h, axis=0, keepdims=True)

        m_old = jnp.max(m_ref[...], axis=0, keepdims=True)
        l_old = jnp.max(l_ref[...], axis=0, keepdims=True)

        m_new = jnp.maximum(m_old, cmax)
        p = jnp.exp2(h - m_new)
        p_sum = jnp.sum(p, axis=0, keepdims=True)
        l_new = l_old * jnp.exp2(m_old - m_new) + p_sum

        row = jax.lax.broadcasted_iota(jnp.int32, (BC, B), 0)
        yloc = y_ref[0] - base
        tsum = jnp.sum(jnp.where(row == yloc, h, 0.0),
                       axis=0, keepdims=True)

        m_ref[...] = jnp.broadcast_to(m_new, m_ref.shape)
        l_ref[...] = jnp.broadcast_to(l_new, l_ref.shape)
        t_ref[...] = t_ref[...] + jnp.broadcast_to(tsum, t_ref.shape)

    base0 = c * (S * BC)
    _chunk(cen_a_ref[...], base0)
    _chunk(cen_b_ref[...], base0 + BC)

    @pl.when(c == NC - 1)
    def _fin():
        m_c = jnp.max(m_ref[...], axis=0, keepdims=True)
        l_c = jnp.max(l_ref[...], axis=0, keepdims=True)
        t_c = jnp.max(t_ref[...], axis=0, keepdims=True)
        loss_row = (m_c + jnp.log2(l_c) - t_c) * _LN2
        corr_row = (t_c == m_c).astype(jnp.float32)
        loss_ref[0] = jnp.sum(loss_row) * (1.0 / B)
        corr_ref[0] = jnp.sum(corr_row) * (1.0 / B)


@jax.jit
def kernel(x, y, centroids):
    y3 = y.astype(jnp.int32).reshape(1, 1, B)
    loss_t, corr_t = pl.pallas_call(
        _main_kernel,
        grid=(NC,),
        in_specs=[
            pl.BlockSpec((B, D // 4), lambda c: (0, 0)),
            pl.BlockSpec((B, D // 4), lambda c: (0, 1)),
            pl.BlockSpec((B, D // 4), lambda c: (0, 2)),
            pl.BlockSpec((B, D // 4), lambda c: (0, 3)),
            pl.BlockSpec((BC, D), lambda c: (c * S, 0)),
            pl.BlockSpec((BC, D), lambda c: (c * S + 1, 0)),
            pl.BlockSpec((1, 1, B), lambda c: (0, 0, 0)),
        ],
        out_specs=(pl.BlockSpec(memory_space=pltpu.SMEM),
                   pl.BlockSpec(memory_space=pltpu.SMEM)),
        out_shape=(jax.ShapeDtypeStruct((1,), jnp.float32),
                   jax.ShapeDtypeStruct((1,), jnp.float32)),
        scratch_shapes=[
            pltpu.VMEM((D, B), jnp.float32),
            pltpu.VMEM((8, B), jnp.float32),
            pltpu.VMEM((8, B), jnp.float32),
            pltpu.VMEM((8, B), jnp.float32),
        ],
        compiler_params=pltpu.CompilerParams(
            dimension_semantics=("arbitrary",),
            vmem_limit_bytes=100 * 1024 * 1024,
        ),
    )(x, x, x, x, centroids, centroids, y3)
    return loss_t[0], corr_t[0]
